# baseline (speedup 1.0000x reference)
import jax
import jax.numpy as jnp
from jax.experimental import pallas as pl
from jax.experimental.pallas import tpu as pltpu

_ROWS = 16384
_COLS = 256
_BLOCK_ROWS = 8192


def _copy_body(t_ref, q_ref, t_out, q_out):
    t_out[...] = t_ref[...]
    q_out[...] = q_ref[...]


def kernel(emb_targets, emb_queries):
    grid = (_ROWS // _BLOCK_ROWS,)
    in_spec = pl.BlockSpec(
        (_BLOCK_ROWS, _COLS), lambda i: (i, 0), pipeline_mode=pl.Buffered(2)
    )
    out_spec = pl.BlockSpec(
        (_BLOCK_ROWS, _COLS), lambda i: (i, 0), pipeline_mode=pl.Buffered(1)
    )
    out_t, out_q = pl.pallas_call(
        _copy_body,
        grid=grid,
        in_specs=[in_spec, in_spec],
        out_specs=[out_spec, out_spec],
        out_shape=[
            jax.ShapeDtypeStruct((_ROWS, _COLS), jnp.float32),
            jax.ShapeDtypeStruct((_ROWS, _COLS), jnp.float32),
        ],
        compiler_params=pltpu.CompilerParams(vmem_limit_bytes=100 * 1024 * 1024),
    )(emb_targets, emb_queries)
    return (out_t, out_q)
